# baseline (speedup 1.0000x reference)
import jax
import jax.numpy as jnp
from jax.experimental import pallas as pl
from jax.experimental.pallas import tpu as pltpu

B = 4
D = 768
N = 1000
T = 2048
WS_CHUNKS = 4
WS_ROWS = D // WS_CHUNKS
WC_SPLITS = (0, 256, 512, 760, N)
WC_CHUNKS = len(WC_SPLITS) - 1


def _body(tok_hbm, emb_hbm, ws_hbm, bs_hbm, wct_hbm, bc_hbm, out_ref,
          tok_s, x_ref, ws_v, wct_v, bs_v, bc_v, sems):
    ws_cps = [
        pltpu.make_async_copy(
            ws_hbm.at[pl.ds(i * WS_ROWS, WS_ROWS), :],
            ws_v.at[pl.ds(i * WS_ROWS, WS_ROWS), :],
            sems.at[i],
        )
        for i in range(WS_CHUNKS)
    ]
    wct_cps = [
        pltpu.make_async_copy(
            wct_hbm.at[pl.ds(WC_SPLITS[i], WC_SPLITS[i + 1] - WC_SPLITS[i]), :],
            wct_v.at[pl.ds(WC_SPLITS[i], WC_SPLITS[i + 1] - WC_SPLITS[i]), :],
            sems.at[WS_CHUNKS + i],
        )
        for i in range(WC_CHUNKS)
    ]
    k = WS_CHUNKS + WC_CHUNKS
    tok_cp = pltpu.make_async_copy(
        tok_hbm.at[:, pl.ds(T - 128, 128)], tok_s, sems.at[k])
    bs_cp = pltpu.make_async_copy(bs_hbm, bs_v, sems.at[k + 1])
    bc_cp = pltpu.make_async_copy(bc_hbm, bc_v, sems.at[k + 2])
    tok_cp.start()
    for c in ws_cps:
        c.start()
    bs_cp.start()
    bc_cp.start()
    for c in wct_cps:
        c.start()
    tok_cp.wait()
    row_cps = [
        pltpu.make_async_copy(
            emb_hbm.at[pl.ds(tok_s[i, 127], 1), :],
            x_ref.at[pl.ds(i, 1), :],
            sems.at[k + 3 + i],
        )
        for i in range(B)
    ]
    for c in row_cps:
        c.start()
    for c in row_cps:
        c.wait()
    for c in ws_cps:
        c.wait()
    bs_cp.wait()
    x = x_ref[...]
    h = jnp.tanh(
        jax.lax.dot_general(x, ws_v[...], (((1,), (0,)), ((), ())),
                            preferred_element_type=jnp.float32)
        + bs_v[...].reshape(1, D)
    )
    y = x + h
    for c in wct_cps:
        c.wait()
    bc_cp.wait()
    out_ref[...] = (
        jax.lax.dot_general(y, wct_v[...], (((1,), (1,)), ((), ())),
                            preferred_element_type=jnp.float32)
        + bc_v[...].reshape(1, N)
    )


def kernel(tokens, embed_table, W_s, b_s, W_c, b_c):
    logits = pl.pallas_call(
        _body,
        in_specs=[pl.BlockSpec(memory_space=pl.ANY)] * 6,
        out_specs=pl.BlockSpec((B, N), lambda: (0, 0)),
        out_shape=jax.ShapeDtypeStruct((B, N), jnp.float32),
        scratch_shapes=[
            pltpu.SMEM((B, 128), jnp.int32),
            pltpu.VMEM((B, D), jnp.float32),
            pltpu.VMEM((D, D), jnp.float32),
            pltpu.VMEM((N, D), jnp.float32),
            pltpu.VMEM((D,), jnp.float32),
            pltpu.VMEM((N,), jnp.float32),
            pltpu.SemaphoreType.DMA((WS_CHUNKS + WC_CHUNKS + 3 + B,)),
        ],
    )(tokens.astype(jnp.int32), embed_table, W_s, b_s, W_c.T, b_c)
    return (logits, None)


# speedup vs baseline: 1.0126x; 1.0126x over previous
import jax
import jax.numpy as jnp
from jax.experimental import pallas as pl
from jax.experimental.pallas import tpu as pltpu

B = 4
D = 768
N = 1000
T = 2048
WS_CHUNKS = 2
WS_ROWS = D // WS_CHUNKS
WC_SPLITS = (0, 512, N)
WC_CHUNKS = len(WC_SPLITS) - 1


def _body(tok_hbm, emb_hbm, ws_hbm, bs_hbm, wct_hbm, bc_hbm, out_ref,
          tok_s, x_ref, ws_v, wct_v, bs_v, bc_v, sems):
    ws_cps = [
        pltpu.make_async_copy(
            ws_hbm.at[pl.ds(i * WS_ROWS, WS_ROWS), :],
            ws_v.at[pl.ds(i * WS_ROWS, WS_ROWS), :],
            sems.at[i],
        )
        for i in range(WS_CHUNKS)
    ]
    wct_cps = [
        pltpu.make_async_copy(
            wct_hbm.at[pl.ds(WC_SPLITS[i], WC_SPLITS[i + 1] - WC_SPLITS[i]), :],
            wct_v.at[pl.ds(WC_SPLITS[i], WC_SPLITS[i + 1] - WC_SPLITS[i]), :],
            sems.at[WS_CHUNKS + i],
        )
        for i in range(WC_CHUNKS)
    ]
    k = WS_CHUNKS + WC_CHUNKS
    tok_cp = pltpu.make_async_copy(
        tok_hbm.at[:, pl.ds(T - 128, 128)], tok_s, sems.at[k])
    bs_cp = pltpu.make_async_copy(bs_hbm, bs_v, sems.at[k + 1])
    bc_cp = pltpu.make_async_copy(bc_hbm, bc_v, sems.at[k + 2])
    tok_cp.start()
    for c in ws_cps:
        c.start()
    bs_cp.start()
    bc_cp.start()
    for c in wct_cps:
        c.start()
    tok_cp.wait()
    row_cps = [
        pltpu.make_async_copy(
            emb_hbm.at[pl.ds(tok_s[i, 127], 1), :],
            x_ref.at[pl.ds(i, 1), :],
            sems.at[k + 3 + i],
        )
        for i in range(B)
    ]
    for c in row_cps:
        c.start()
    for c in row_cps:
        c.wait()
    for c in ws_cps:
        c.wait()
    bs_cp.wait()
    x = x_ref[...]
    h = jnp.tanh(
        jax.lax.dot_general(x, ws_v[...], (((1,), (0,)), ((), ())),
                            preferred_element_type=jnp.float32)
        + bs_v[...].reshape(1, D)
    )
    y = x + h
    for c in wct_cps:
        c.wait()
    bc_cp.wait()
    out_ref[...] = (
        jax.lax.dot_general(y, wct_v[...], (((1,), (1,)), ((), ())),
                            preferred_element_type=jnp.float32)
        + bc_v[...].reshape(1, N)
    )


def kernel(tokens, embed_table, W_s, b_s, W_c, b_c):
    logits = pl.pallas_call(
        _body,
        in_specs=[pl.BlockSpec(memory_space=pl.ANY)] * 6,
        out_specs=pl.BlockSpec((B, N), lambda: (0, 0)),
        out_shape=jax.ShapeDtypeStruct((B, N), jnp.float32),
        scratch_shapes=[
            pltpu.SMEM((B, 128), jnp.int32),
            pltpu.VMEM((B, D), jnp.float32),
            pltpu.VMEM((D, D), jnp.float32),
            pltpu.VMEM((N, D), jnp.float32),
            pltpu.VMEM((D,), jnp.float32),
            pltpu.VMEM((N,), jnp.float32),
            pltpu.SemaphoreType.DMA((WS_CHUNKS + WC_CHUNKS + 3 + B,)),
        ],
    )(tokens.astype(jnp.int32), embed_table, W_s, b_s, W_c.T, b_c)
    return (logits, None)
